# baseline (speedup 1.0000x reference)
import functools
from math import comb

import jax
import jax.numpy as jnp
import numpy as np
from jax import lax
from jax.experimental import pallas as pl
from jax.experimental.pallas import tpu as pltpu
from jax.experimental.pallas import tpu_sc as plsc

N = 10000
E = 320000
K = 10
L = 16
NC = 2
NT = 16
NP = 10112
NPT = NP // NT
CHUNK = 128
NCHUNK = 160
EPT = NCHUNK * CHUNK
EPAD = NT * EPT
NJUNK = 16
NV = CHUNK // L
NCH2 = NCHUNK + 8
NBUF = 8


def _basis_matrix():
    B = np.zeros((K + 1, K + 1))
    for j in range(K + 1):
        p = np.array([1.0])
        for _ in range(j):
            p = np.convolve(p, [1.0, -1.0])
        for _ in range(K - j):
            p = np.convolve(p, [1.0, 1.0])
        B[j, : len(p)] = p
    return B


_BM = _basis_matrix().astype(np.float32)
_BINOM = np.array([comb(K, j) / 2.0**K for j in range(K + 1)], np.float32)


def _tc1_body(x_ref, w_ref, deg_ref, o_ref, dis_ref, dis2_ref):
    o_ref[...] = jnp.dot(x_ref[...], w_ref[...],
                         preferred_element_type=jnp.float32)
    d = deg_ref[...]
    pos = d > 0.0
    dis_ref[...] = jnp.where(pos, lax.rsqrt(d), 0.0)
    dis2_ref[...] = jnp.where(pos, 1.0 / d, 0.0)


def _tc2_body(o_ref, b_ref, w_ref, z_ref):
    h = jnp.maximum(o_ref[...] + b_ref[...], 0.0)
    z_ref[...] = jnp.dot(h, w_ref[...], preferred_element_type=jnp.float32)


def _tc3_body(o_ref, b_ref, w_ref, c_ref, y_ref):
    h = jnp.maximum(o_ref[...] + b_ref[...], 0.0)
    y_ref[...] = jnp.sum(h * w_ref[...], axis=1, keepdims=True) + c_ref[...]


def _rows_loop(n, unroll, fn):
    def body(k, carry):
        for u in range(unroll):
            fn(k * unroll + u)
        return carry
    lax.fori_loop(0, n // unroll, body, 0)


def _sc_deg_body(row_hbm, deg_hbm, row_v, ones_v, zb_v, t_v, tf_v,
                 ss0, ss1, T):
    cid = lax.axis_index("c")
    sid = lax.axis_index("s")
    base = sid * NPT
    pltpu.sync_copy(row_hbm.at[sid], row_v)

    ones16 = jnp.ones((L,), jnp.float32)
    zeros16 = jnp.zeros((L,), jnp.float32)
    for u in range(CHUNK):
        ones_v[u] = ones16

    def zero_row(i):
        zb_v[i] = zeros16
    _rows_loop(NPT, 8, zero_row)

    pltpu.sync_copy(zb_v, T.at[pl.ds(base, NPT)])
    plsc.subcore_barrier()

    def deg_pair(k, carry):
        j0 = k * 2
        j1 = j0 + 1
        d0 = pltpu.async_copy(ones_v, T.at[row_v.at[j0]], ss0, add=True)
        d1 = pltpu.async_copy(ones_v, T.at[row_v.at[j1]], ss1, add=True)
        d0.wait()
        d1.wait()
        return carry

    lax.fori_loop(0, NCHUNK // 2, deg_pair, 0)
    plsc.subcore_barrier()
    pltpu.sync_copy(T.at[pl.ds(base, NPT)], t_v)

    def flat_row(i):
        tf_v[pl.ds(i * L, L)] = t_v[i]
    _rows_loop(NPT, 8, flat_row)
    pltpu.sync_copy(tf_v, deg_hbm.at[pl.ds((cid * NP + base) * L, NPT * L)])


def _sc_bern_body(z_hbm, row_hbm, col_hbm, c_hbm, dis_hbm, dis2_hbm,
                  out_hbm,
                  row_v, col_v, c_v, z_v, dis_v, dis2_v, zb_v, t_v,
                  ga0, ga1, ga2, ga3, ga4, ga5, ga6, ga7,
                  sg0, sg1, sg2, sg3, sg4, sg5, sg6, sg7,
                  ss0, ss1, ss2, ss3, ss4, ss5, ss6, ss7, Gd, T):
    cid = lax.axis_index("c")
    sid = lax.axis_index("s")
    base = sid * NPT
    zoff = (cid * NP + base) * L
    G = Gd.at[cid]

    pltpu.sync_copy(row_hbm.at[sid], row_v)
    pltpu.sync_copy(col_hbm.at[sid], col_v)
    pltpu.sync_copy(c_hbm, c_v)
    pltpu.sync_copy(z_hbm.at[pl.ds(zoff, NPT * L)], z_v)
    pltpu.sync_copy(dis_hbm.at[pl.ds(base * L, NPT * L)], dis_v)
    pltpu.sync_copy(dis2_hbm.at[pl.ds(base * L, NPT * L)], dis2_v)

    zeros16 = jnp.zeros((L,), jnp.float32)

    def zero_row(i):
        zb_v[i] = zeros16
    _rows_loop(NPT, 8, zero_row)
    pltpu.sync_copy(zb_v, T.at[pl.ds(base, NPT)])

    cK = c_v[pl.ds(K * L, L)]

    def init_row(i):
        t_v[i] = cK * (dis_v[pl.ds(i * L, L)] * z_v[pl.ds(i * L, L)])
    _rows_loop(NPT, 8, init_row)
    pltpu.sync_copy(t_v, G.at[pl.ds(base, NPT)])
    plsc.subcore_barrier()

    gas = (ga0, ga1, ga2, ga3, ga4, ga5, ga6, ga7)
    sgs = (sg0, sg1, sg2, sg3, sg4, sg5, sg6, sg7)
    sss = (ss0, ss1, ss2, ss3, ss4, ss5, ss6, ss7)
    for m in range(K - 1, -1, -1):
        for b in range(NBUF):
            pltpu.async_copy(G.at[row_v.at[b]], gas[b], sgs[b])

        def edge_ring(k, carry):
            j = k * NBUF
            descs = []
            for b in range(NBUF):
                pltpu.make_async_copy(
                    G.at[row_v.at[j + b]], gas[b], sgs[b]).wait()
                descs.append(pltpu.async_copy(
                    gas[b], T.at[col_v.at[j + b]], sss[b], add=True))
            for b in range(NBUF):
                descs[b].wait()
                pltpu.async_copy(
                    G.at[row_v.at[j + b + NBUF]], gas[b], sgs[b])
            return carry

        lax.fori_loop(0, NCHUNK // NBUF, edge_ring, 0)
        for b in range(NBUF):
            pltpu.make_async_copy(
                G.at[row_v.at[NCHUNK + b]], gas[b], sgs[b]).wait()
        plsc.subcore_barrier()

        pltpu.sync_copy(T.at[pl.ds(base, NPT)], t_v)
        cm = c_v[pl.ds(m * L, L)]
        if m > 0:
            def post_row(i):
                s = pl.ds(i * L, L)
                t_v[i] = (t_v[i] * dis2_v[s]
                          + cm * (dis_v[s] * z_v[s]))
            _rows_loop(NPT, 8, post_row)
            pltpu.sync_copy(t_v, G.at[pl.ds(base, NPT)])
            pltpu.sync_copy(zb_v, T.at[pl.ds(base, NPT)])
            plsc.subcore_barrier()
        else:
            def final_row(i):
                s = pl.ds(i * L, L)
                z_v[s] = t_v[i] * dis_v[s] + cm * z_v[s]
            _rows_loop(NPT, 8, final_row)
            pltpu.sync_copy(z_v, out_hbm.at[pl.ds(zoff, NPT * L)])


_sc_mesh = plsc.VectorSubcoreMesh(core_axis_name="c", subcore_axis_name="s",
                                  num_cores=NC, num_subcores=NT)

_sc_params = pltpu.CompilerParams(use_tc_tiling_on_sc=False)

_sc_deg = functools.partial(
    pl.kernel,
    out_type=jax.ShapeDtypeStruct((NC * NP * L,), jnp.float32),
    mesh=_sc_mesh,
    compiler_params=_sc_params,
    scratch_types=[
        pltpu.VMEM((NCH2, CHUNK), jnp.int32),
        pltpu.VMEM((CHUNK, L), jnp.float32),
        pltpu.VMEM((NPT, L), jnp.float32),
        pltpu.VMEM((NPT, L), jnp.float32),
        pltpu.VMEM((NPT * L,), jnp.float32),
        pltpu.SemaphoreType.DMA,
        pltpu.SemaphoreType.DMA,
        pltpu.VMEM_SHARED((NP, L), jnp.float32),
    ],
)(_sc_deg_body)

_sc_bern = functools.partial(
    pl.kernel,
    out_type=jax.ShapeDtypeStruct((NC * NP * L,), jnp.float32),
    mesh=_sc_mesh,
    compiler_params=_sc_params,
    scratch_types=[
        pltpu.VMEM((NCH2, CHUNK), jnp.int32),
        pltpu.VMEM((NCH2, CHUNK), jnp.int32),
        pltpu.VMEM((16 * L,), jnp.float32),
        pltpu.VMEM((NPT * L,), jnp.float32),
        pltpu.VMEM((NPT * L,), jnp.float32),
        pltpu.VMEM((NPT * L,), jnp.float32),
        pltpu.VMEM((NPT, L), jnp.float32),
        pltpu.VMEM((NPT, L), jnp.float32),
        pltpu.VMEM((CHUNK, L), jnp.float32),
        pltpu.VMEM((CHUNK, L), jnp.float32),
        pltpu.VMEM((CHUNK, L), jnp.float32),
        pltpu.VMEM((CHUNK, L), jnp.float32),
        pltpu.VMEM((CHUNK, L), jnp.float32),
        pltpu.VMEM((CHUNK, L), jnp.float32),
        pltpu.VMEM((CHUNK, L), jnp.float32),
        pltpu.VMEM((CHUNK, L), jnp.float32),
        pltpu.SemaphoreType.DMA,
        pltpu.SemaphoreType.DMA,
        pltpu.SemaphoreType.DMA,
        pltpu.SemaphoreType.DMA,
        pltpu.SemaphoreType.DMA,
        pltpu.SemaphoreType.DMA,
        pltpu.SemaphoreType.DMA,
        pltpu.SemaphoreType.DMA,
        pltpu.SemaphoreType.DMA,
        pltpu.SemaphoreType.DMA,
        pltpu.SemaphoreType.DMA,
        pltpu.SemaphoreType.DMA,
        pltpu.SemaphoreType.DMA,
        pltpu.SemaphoreType.DMA,
        pltpu.SemaphoreType.DMA,
        pltpu.SemaphoreType.DMA,
        pltpu.HBM((NC, NP, L), jnp.float32),
        pltpu.VMEM_SHARED((NP, L), jnp.float32),
    ],
)(_sc_bern_body)


def _split(z):
    zp = jnp.pad(z, ((0, NP - N), (0, 0)))
    return zp.reshape(NP, NC, L).transpose(1, 0, 2).reshape(-1)


def _merge(zf):
    return zf.reshape(NC, NP, L).transpose(1, 0, 2).reshape(NP, NC * L)[:N]


def kernel(x, edge_index, coe, W1, b1, W2, b2, fc_W, fc_b):
    a = jax.nn.relu(coe) * jnp.asarray(_BINOM)
    c = a @ jnp.asarray(_BM)
    cb = jnp.zeros((16, L), jnp.float32).at[: K + 1, :].set(
        jnp.broadcast_to(c[:, None], (K + 1, L))).reshape(-1)

    pad = jnp.arange(EPAD - E, dtype=jnp.int32) % NJUNK + N
    rows_f = jnp.concatenate([edge_index[0], pad])
    cols_f = jnp.concatenate([edge_index[1], pad])
    dummy = jnp.broadcast_to(
        (jnp.arange(NBUF * CHUNK, dtype=jnp.int32) % NJUNK + N)[None],
        (NT, NBUF * CHUNK))
    rows_c = jnp.concatenate(
        [rows_f.reshape(NT, EPT), dummy], axis=1).reshape(NT, NCH2, CHUNK)
    cols_c = jnp.concatenate(
        [cols_f.reshape(NT, EPT), dummy], axis=1).reshape(NT, NCH2, CHUNK)

    degf = _sc_deg(rows_c)
    deg = degf[: NP * L].reshape(NP, L)

    z1, dis_b, dis2_b = pl.pallas_call(
        _tc1_body,
        out_shape=(
            jax.ShapeDtypeStruct((N, 32), jnp.float32),
            jax.ShapeDtypeStruct((NP, L), jnp.float32),
            jax.ShapeDtypeStruct((NP, L), jnp.float32),
        ),
    )(x, W1, deg)
    dis_f = dis_b.reshape(-1)
    dis2_f = dis2_b.reshape(-1)

    o1 = _merge(_sc_bern(_split(z1), rows_c, cols_c, cb, dis_f, dis2_f))

    z2 = pl.pallas_call(
        _tc2_body,
        out_shape=jax.ShapeDtypeStruct((N, 32), jnp.float32),
    )(o1, b1.reshape(1, 32), W2)

    o2 = _merge(_sc_bern(_split(z2), rows_c, cols_c, cb, dis_f, dis2_f))

    y = pl.pallas_call(
        _tc3_body,
        out_shape=jax.ShapeDtypeStruct((N, 1), jnp.float32),
    )(o2, b2.reshape(1, 32), fc_W.reshape(1, 32), fc_b.reshape(1, 1))

    return y


# speedup vs baseline: 1.9603x; 1.9603x over previous
import functools
from math import comb

import jax
import jax.numpy as jnp
import numpy as np
from jax import lax
from jax.experimental import pallas as pl
from jax.experimental.pallas import tpu as pltpu
from jax.experimental.pallas import tpu_sc as plsc

N = 10000
E = 320000
K = 10
L = 16
NC = 2
NT = 16
NP = 10112
NPT = NP // NT
CHUNK = 128
NCHUNK = 160
EPT = NCHUNK * CHUNK
EPAD = NT * EPT
NJUNK = 16
NV = CHUNK // L
NCH2 = NCHUNK + 8
NBUF = 8


def _basis_matrix():
    B = np.zeros((K + 1, K + 1))
    for j in range(K + 1):
        p = np.array([1.0])
        for _ in range(j):
            p = np.convolve(p, [1.0, -1.0])
        for _ in range(K - j):
            p = np.convolve(p, [1.0, 1.0])
        B[j, : len(p)] = p
    return B


_BM = _basis_matrix().astype(np.float32)
_BINOM = np.array([comb(K, j) / 2.0**K for j in range(K + 1)], np.float32)


def _tc1_body(x_ref, w_ref, deg_ref, o_ref, dis_ref, dis2_ref):
    o_ref[...] = jnp.dot(x_ref[...], w_ref[...],
                         preferred_element_type=jnp.float32)
    d = deg_ref[...]
    pos = d > 0.0
    dis_ref[...] = jnp.where(pos, lax.rsqrt(d), 0.0)
    dis2_ref[...] = jnp.where(pos, 1.0 / d, 0.0)


def _tc2_body(o_ref, b_ref, w_ref, z_ref):
    h = jnp.maximum(o_ref[...] + b_ref[...], 0.0)
    z_ref[...] = jnp.dot(h, w_ref[...], preferred_element_type=jnp.float32)


def _tc3_body(o_ref, b_ref, w_ref, c_ref, y_ref):
    h = jnp.maximum(o_ref[...] + b_ref[...], 0.0)
    y_ref[...] = jnp.sum(h * w_ref[...], axis=1, keepdims=True) + c_ref[...]


def _rows_loop(n, unroll, fn):
    def body(k, carry):
        for u in range(unroll):
            fn(k * unroll + u)
        return carry
    lax.fori_loop(0, n // unroll, body, 0)


def _sc_deg_body(row_hbm, deg_hbm, row_v, ones_v, zb_v, t_v, tf_v,
                 ss0, ss1, T):
    cid = lax.axis_index("c")
    sid = lax.axis_index("s")
    base = sid * NPT
    pltpu.sync_copy(row_hbm.at[sid], row_v)

    ones16 = jnp.ones((L,), jnp.float32)
    zeros16 = jnp.zeros((L,), jnp.float32)
    for u in range(CHUNK):
        ones_v[u] = ones16

    def zero_row(i):
        zb_v[i] = zeros16
    _rows_loop(NPT, 8, zero_row)

    pltpu.sync_copy(zb_v, T.at[pl.ds(base, NPT)])
    plsc.subcore_barrier()

    def deg_pair(k, carry):
        j0 = k * 2
        j1 = j0 + 1
        d0 = pltpu.async_copy(ones_v, T.at[row_v.at[j0]], ss0, add=True)
        d1 = pltpu.async_copy(ones_v, T.at[row_v.at[j1]], ss1, add=True)
        d0.wait()
        d1.wait()
        return carry

    lax.fori_loop(0, NCHUNK // 2, deg_pair, 0)
    plsc.subcore_barrier()
    pltpu.sync_copy(T.at[pl.ds(base, NPT)], t_v)

    def flat_row(i):
        tf_v[pl.ds(i * L, L)] = t_v[i]
    _rows_loop(NPT, 8, flat_row)
    pltpu.sync_copy(tf_v, deg_hbm.at[pl.ds((cid * NP + base) * L, NPT * L)])


def _sc_bern_body(z_hbm, row_hbm, col_hbm, c_hbm, dis_hbm, dis2_hbm,
                  out_hbm,
                  row_v, col_v, c_v, z_v, dis_v, dis2_v, zb_v, t_v,
                  ga0, ga1, ga2, ga3, ga4, ga5, ga6, ga7,
                  sg0, sg1, sg2, sg3, sg4, sg5, sg6, sg7,
                  ss0, ss1, ss2, ss3, ss4, ss5, ss6, ss7, G, T):
    cid = lax.axis_index("c")
    sid = lax.axis_index("s")
    base = sid * NPT
    zoff = (cid * NP + base) * L

    pltpu.sync_copy(row_hbm.at[sid], row_v)
    pltpu.sync_copy(col_hbm.at[sid], col_v)
    pltpu.sync_copy(c_hbm, c_v)
    pltpu.sync_copy(z_hbm.at[pl.ds(zoff, NPT * L)], z_v)
    pltpu.sync_copy(dis_hbm.at[pl.ds(base * L, NPT * L)], dis_v)
    pltpu.sync_copy(dis2_hbm.at[pl.ds(base * L, NPT * L)], dis2_v)

    zeros16 = jnp.zeros((L,), jnp.float32)

    def zero_row(i):
        zb_v[i] = zeros16
    _rows_loop(NPT, 8, zero_row)
    pltpu.sync_copy(zb_v, T.at[pl.ds(base, NPT)])

    cK = c_v[pl.ds(K * L, L)]

    def init_row(i):
        t_v[i] = cK * (dis_v[pl.ds(i * L, L)] * z_v[pl.ds(i * L, L)])
    _rows_loop(NPT, 8, init_row)
    pltpu.sync_copy(t_v, G.at[pl.ds(base, NPT)])
    plsc.subcore_barrier()

    gas = (ga0, ga1, ga2, ga3, ga4, ga5, ga6, ga7)
    sgs = (sg0, sg1, sg2, sg3, sg4, sg5, sg6, sg7)
    sss = (ss0, ss1, ss2, ss3, ss4, ss5, ss6, ss7)
    for m in range(K - 1, -1, -1):
        for b in range(NBUF):
            pltpu.async_copy(G.at[row_v.at[b]], gas[b], sgs[b])

        def edge_ring(k, carry):
            j = k * NBUF
            descs = []
            for b in range(NBUF):
                pltpu.make_async_copy(
                    G.at[row_v.at[j + b]], gas[b], sgs[b]).wait()
                descs.append(pltpu.async_copy(
                    gas[b], T.at[col_v.at[j + b]], sss[b], add=True))
            for b in range(NBUF):
                descs[b].wait()
                pltpu.async_copy(
                    G.at[row_v.at[j + b + NBUF]], gas[b], sgs[b])
            return carry

        lax.fori_loop(0, NCHUNK // NBUF, edge_ring, 0)
        for b in range(NBUF):
            pltpu.make_async_copy(
                G.at[row_v.at[NCHUNK + b]], gas[b], sgs[b]).wait()
        plsc.subcore_barrier()

        pltpu.sync_copy(T.at[pl.ds(base, NPT)], t_v)
        cm = c_v[pl.ds(m * L, L)]
        if m > 0:
            def post_row(i):
                s = pl.ds(i * L, L)
                t_v[i] = (t_v[i] * dis2_v[s]
                          + cm * (dis_v[s] * z_v[s]))
            _rows_loop(NPT, 8, post_row)
            pltpu.sync_copy(t_v, G.at[pl.ds(base, NPT)])
            pltpu.sync_copy(zb_v, T.at[pl.ds(base, NPT)])
            plsc.subcore_barrier()
        else:
            def final_row(i):
                s = pl.ds(i * L, L)
                z_v[s] = t_v[i] * dis_v[s] + cm * z_v[s]
            _rows_loop(NPT, 8, final_row)
            pltpu.sync_copy(z_v, out_hbm.at[pl.ds(zoff, NPT * L)])


_sc_mesh = plsc.VectorSubcoreMesh(core_axis_name="c", subcore_axis_name="s",
                                  num_cores=NC, num_subcores=NT)

_sc_params = pltpu.CompilerParams(use_tc_tiling_on_sc=False)

_sc_deg = functools.partial(
    pl.kernel,
    out_type=jax.ShapeDtypeStruct((NC * NP * L,), jnp.float32),
    mesh=_sc_mesh,
    compiler_params=_sc_params,
    scratch_types=[
        pltpu.VMEM((NCH2, CHUNK), jnp.int32),
        pltpu.VMEM((CHUNK, L), jnp.float32),
        pltpu.VMEM((NPT, L), jnp.float32),
        pltpu.VMEM((NPT, L), jnp.float32),
        pltpu.VMEM((NPT * L,), jnp.float32),
        pltpu.SemaphoreType.DMA,
        pltpu.SemaphoreType.DMA,
        pltpu.VMEM_SHARED((NP, L), jnp.float32),
    ],
)(_sc_deg_body)

_sc_bern = functools.partial(
    pl.kernel,
    out_type=jax.ShapeDtypeStruct((NC * NP * L,), jnp.float32),
    mesh=_sc_mesh,
    compiler_params=_sc_params,
    scratch_types=[
        pltpu.VMEM((NCH2, CHUNK), jnp.int32),
        pltpu.VMEM((NCH2, CHUNK), jnp.int32),
        pltpu.VMEM((16 * L,), jnp.float32),
        pltpu.VMEM((NPT * L,), jnp.float32),
        pltpu.VMEM((NPT * L,), jnp.float32),
        pltpu.VMEM((NPT * L,), jnp.float32),
        pltpu.VMEM((NPT, L), jnp.float32),
        pltpu.VMEM((NPT, L), jnp.float32),
        pltpu.VMEM((CHUNK, L), jnp.float32),
        pltpu.VMEM((CHUNK, L), jnp.float32),
        pltpu.VMEM((CHUNK, L), jnp.float32),
        pltpu.VMEM((CHUNK, L), jnp.float32),
        pltpu.VMEM((CHUNK, L), jnp.float32),
        pltpu.VMEM((CHUNK, L), jnp.float32),
        pltpu.VMEM((CHUNK, L), jnp.float32),
        pltpu.VMEM((CHUNK, L), jnp.float32),
        pltpu.SemaphoreType.DMA,
        pltpu.SemaphoreType.DMA,
        pltpu.SemaphoreType.DMA,
        pltpu.SemaphoreType.DMA,
        pltpu.SemaphoreType.DMA,
        pltpu.SemaphoreType.DMA,
        pltpu.SemaphoreType.DMA,
        pltpu.SemaphoreType.DMA,
        pltpu.SemaphoreType.DMA,
        pltpu.SemaphoreType.DMA,
        pltpu.SemaphoreType.DMA,
        pltpu.SemaphoreType.DMA,
        pltpu.SemaphoreType.DMA,
        pltpu.SemaphoreType.DMA,
        pltpu.SemaphoreType.DMA,
        pltpu.SemaphoreType.DMA,
        pltpu.VMEM_SHARED((NP, L), jnp.float32),
        pltpu.VMEM_SHARED((NP, L), jnp.float32),
    ],
)(_sc_bern_body)


def _split(z):
    zp = jnp.pad(z, ((0, NP - N), (0, 0)))
    return zp.reshape(NP, NC, L).transpose(1, 0, 2).reshape(-1)


def _merge(zf):
    return zf.reshape(NC, NP, L).transpose(1, 0, 2).reshape(NP, NC * L)[:N]


def kernel(x, edge_index, coe, W1, b1, W2, b2, fc_W, fc_b):
    a = jax.nn.relu(coe) * jnp.asarray(_BINOM)
    c = a @ jnp.asarray(_BM)
    cb = jnp.zeros((16, L), jnp.float32).at[: K + 1, :].set(
        jnp.broadcast_to(c[:, None], (K + 1, L))).reshape(-1)

    pad = jnp.arange(EPAD - E, dtype=jnp.int32) % NJUNK + N
    rows_f = jnp.concatenate([edge_index[0], pad])
    cols_f = jnp.concatenate([edge_index[1], pad])
    dummy = jnp.broadcast_to(
        (jnp.arange(NBUF * CHUNK, dtype=jnp.int32) % NJUNK + N)[None],
        (NT, NBUF * CHUNK))
    rows_c = jnp.concatenate(
        [rows_f.reshape(NT, EPT), dummy], axis=1).reshape(NT, NCH2, CHUNK)
    cols_c = jnp.concatenate(
        [cols_f.reshape(NT, EPT), dummy], axis=1).reshape(NT, NCH2, CHUNK)

    degf = _sc_deg(rows_c)
    deg = degf[: NP * L].reshape(NP, L)

    z1, dis_b, dis2_b = pl.pallas_call(
        _tc1_body,
        out_shape=(
            jax.ShapeDtypeStruct((N, 32), jnp.float32),
            jax.ShapeDtypeStruct((NP, L), jnp.float32),
            jax.ShapeDtypeStruct((NP, L), jnp.float32),
        ),
    )(x, W1, deg)
    dis_f = dis_b.reshape(-1)
    dis2_f = dis2_b.reshape(-1)

    o1 = _merge(_sc_bern(_split(z1), rows_c, cols_c, cb, dis_f, dis2_f))

    z2 = pl.pallas_call(
        _tc2_body,
        out_shape=jax.ShapeDtypeStruct((N, 32), jnp.float32),
    )(o1, b1.reshape(1, 32), W2)

    o2 = _merge(_sc_bern(_split(z2), rows_c, cols_c, cb, dis_f, dis2_f))

    y = pl.pallas_call(
        _tc3_body,
        out_shape=jax.ShapeDtypeStruct((N, 1), jnp.float32),
    )(o2, b2.reshape(1, 32), fc_W.reshape(1, 32), fc_b.reshape(1, 1))

    return y
